# Initial kernel scaffold; baseline (speedup 1.0000x reference)
import functools

import jax
import jax.numpy as jnp
from jax import lax
from jax.experimental import pallas as pl
from jax.experimental.pallas import tpu as pltpu
from jax.experimental.pallas import tpu_sc as plsc

N = 10000
D = 128
E = 320000

NC = 2
NS = 16
NW = NC * NS

CHUNK = 128
ROWS_PT = 79
E_PAD = NW * ROWS_PT * CHUNK
N_ACC = N + 16
INIT_PT = N_ACC // NS
OUT_PT = N // NS
DEG_W = 16

_MESH = plsc.VectorSubcoreMesh(
    core_axis_name="c", subcore_axis_name="s", num_cores=NC, num_subcores=NS
)


@functools.partial(
    pl.kernel,
    out_type=jax.ShapeDtypeStruct((NC, N, DEG_W), jnp.float32),
    mesh=_MESH,
    scratch_types=[
        pltpu.VMEM((ROWS_PT, CHUNK), jnp.int32),
        pltpu.VMEM((CHUNK, DEG_W), jnp.float32),
        pltpu.VMEM_SHARED((N_ACC, DEG_W), jnp.float32),
    ],
)
def _deg_kernel(dst_hbm, ones_hbm, zeros_hbm, out_hbm, dst_v, ones_v, acc):
    c = lax.axis_index("c")
    s = lax.axis_index("s")
    wid = c * NS + s
    pltpu.sync_copy(zeros_hbm, acc.at[pl.ds(s * INIT_PT, INIT_PT)])
    pltpu.sync_copy(dst_hbm.at[pl.ds(wid * ROWS_PT, ROWS_PT)], dst_v)
    pltpu.sync_copy(ones_hbm, ones_v)
    plsc.subcore_barrier()

    def body(j, carry):
        pltpu.sync_copy(ones_v, acc.at[dst_v.at[j]], add=True)
        return carry

    lax.fori_loop(0, ROWS_PT, body, 0)
    plsc.subcore_barrier()
    pltpu.sync_copy(
        acc.at[pl.ds(s * OUT_PT, OUT_PT)], out_hbm.at[c, pl.ds(s * OUT_PT, OUT_PT)]
    )


@functools.partial(
    pl.kernel,
    out_type=jax.ShapeDtypeStruct((NC, N, D), jnp.float32),
    mesh=_MESH,
    scratch_types=[
        pltpu.VMEM((ROWS_PT, CHUNK), jnp.int32),
        pltpu.VMEM((ROWS_PT, CHUNK), jnp.int32),
        pltpu.VMEM((CHUNK, D), jnp.float32),
        pltpu.VMEM_SHARED((N_ACC, D), jnp.float32),
        pltpu.SemaphoreType.DMA,
    ],
)
def _prop_kernel(g_hbm, src_hbm, dst_hbm, zeros_hbm, out_hbm, src_v, dst_v, rows_v, acc, sem):
    c = lax.axis_index("c")
    s = lax.axis_index("s")
    wid = c * NS + s
    pltpu.sync_copy(zeros_hbm, acc.at[pl.ds(s * INIT_PT, INIT_PT)])
    pltpu.sync_copy(src_hbm.at[pl.ds(wid * ROWS_PT, ROWS_PT)], src_v)
    pltpu.sync_copy(dst_hbm.at[pl.ds(wid * ROWS_PT, ROWS_PT)], dst_v)
    plsc.subcore_barrier()

    def body(j, carry):
        pltpu.async_copy(g_hbm.at[src_v.at[j]], rows_v, sem).wait()
        pltpu.sync_copy(rows_v, acc.at[dst_v.at[j]], add=True)
        return carry

    lax.fori_loop(0, ROWS_PT, body, 0)
    plsc.subcore_barrier()
    pltpu.sync_copy(
        acc.at[pl.ds(s * OUT_PT, OUT_PT)], out_hbm.at[c, pl.ds(s * OUT_PT, OUT_PT)]
    )


_R = 1000


def _k1_body(x_ref, degp_ref, w_ref, g_ref, dinv_ref):
    dp = degp_ref[0] + degp_ref[1]
    deg = jnp.sum(dp, axis=1, keepdims=True) * (1.0 / DEG_W) + 1.0
    dinv = lax.rsqrt(deg)
    h = jnp.dot(x_ref[...], w_ref[...], preferred_element_type=jnp.float32)
    g_ref[...] = h * dinv
    dinv_ref[...] = dinv


def _k1_call(x, degp, W1):
    grid = (N // _R,)
    return pl.pallas_call(
        _k1_body,
        grid=grid,
        in_specs=[
            pl.BlockSpec((_R, D), lambda i: (i, 0)),
            pl.BlockSpec((NC, _R, DEG_W), lambda i: (0, i, 0)),
            pl.BlockSpec((D, D), lambda i: (0, 0)),
        ],
        out_specs=[
            pl.BlockSpec((_R, D), lambda i: (i, 0)),
            pl.BlockSpec((_R, 1), lambda i: (i, 0)),
        ],
        out_shape=[
            jax.ShapeDtypeStruct((N, D), jnp.float32),
            jax.ShapeDtypeStruct((N, 1), jnp.float32),
        ],
    )(x, degp, W1)


def _k2_body(a_ref, g_ref, dinv_ref, b_ref, w_ref, out_ref):
    dinv = dinv_ref[...]
    z = (a_ref[0] + a_ref[1] + g_ref[...]) * dinv + b_ref[...]
    z = jnp.maximum(z, 0.0)
    out_ref[...] = jnp.dot(z, w_ref[...], preferred_element_type=jnp.float32) * dinv


def _k2_call(A, g1, dinv, b1, W2):
    grid = (N // _R,)
    return pl.pallas_call(
        _k2_body,
        grid=grid,
        in_specs=[
            pl.BlockSpec((NC, _R, D), lambda i: (0, i, 0)),
            pl.BlockSpec((_R, D), lambda i: (i, 0)),
            pl.BlockSpec((_R, 1), lambda i: (i, 0)),
            pl.BlockSpec((1, D), lambda i: (0, 0)),
            pl.BlockSpec((D, D), lambda i: (0, 0)),
        ],
        out_specs=pl.BlockSpec((_R, D), lambda i: (i, 0)),
        out_shape=jax.ShapeDtypeStruct((N, D), jnp.float32),
    )(A, g1, dinv, b1, W2)


def _k3_body(a_ref, g_ref, dinv_ref, b_ref, out_ref):
    out_ref[...] = (a_ref[0] + a_ref[1] + g_ref[...]) * dinv_ref[...] + b_ref[...]


def _k3_call(A, g2, dinv, b2):
    grid = (N // _R,)
    return pl.pallas_call(
        _k3_body,
        grid=grid,
        in_specs=[
            pl.BlockSpec((NC, _R, D), lambda i: (0, i, 0)),
            pl.BlockSpec((_R, D), lambda i: (i, 0)),
            pl.BlockSpec((_R, 1), lambda i: (i, 0)),
            pl.BlockSpec((1, D), lambda i: (0, 0)),
        ],
        out_specs=pl.BlockSpec((_R, D), lambda i: (i, 0)),
        out_shape=jax.ShapeDtypeStruct((N, D), jnp.float32),
    )(A, g2, dinv, b2)


def kernel(x, edge_index, W1, b1, W2, b2):
    src = edge_index[0].astype(jnp.int32)
    dst = edge_index[1].astype(jnp.int32)
    pad = E_PAD - E
    src_p = jnp.concatenate([src, jnp.zeros((pad,), jnp.int32)]).reshape(-1, CHUNK)
    dst_p = jnp.concatenate([dst, jnp.full((pad,), N, jnp.int32)]).reshape(-1, CHUNK)

    ones16 = jnp.ones((CHUNK, DEG_W), jnp.float32)
    zeros16 = jnp.zeros((INIT_PT, DEG_W), jnp.float32)
    zeros128 = jnp.zeros((INIT_PT, D), jnp.float32)

    degp = _deg_kernel(dst_p, ones16, zeros16)
    g1, dinv = _k1_call(x, degp, W1)
    A1 = _prop_kernel(g1, src_p, dst_p, zeros128)
    g2 = _k2_call(A1, g1, dinv, jnp.reshape(b1, (1, D)), W2)
    A2 = _prop_kernel(g2, src_p, dst_p, zeros128)
    out = _k3_call(A2, g2, dinv, jnp.reshape(b2, (1, D)))
    return out


# speedup vs baseline: 8.7952x; 8.7952x over previous
import functools

import jax
import jax.numpy as jnp
from jax import lax
from jax.experimental import pallas as pl
from jax.experimental.pallas import tpu as pltpu
from jax.experimental.pallas import tpu_sc as plsc

N = 10000
D = 128
E = 320000

NC = 2
NS = 16
NW = NC * NS

CHUNK = 128
ROWS_PT = 80
E_PAD = NW * ROWS_PT * CHUNK
N_ACC = 10240
INIT_PT = N_ACC // NS
OUT_PT = INIT_PT
DEG_W = 16

_MESH = plsc.VectorSubcoreMesh(
    core_axis_name="c", subcore_axis_name="s", num_cores=NC, num_subcores=NS
)


def _deg_body(dst_hbm, ones_hbm, zeros_hbm, out_hbm, dst_v, ones_v, acc):
    c = lax.axis_index("c")
    s = lax.axis_index("s")
    wid = c * NS + s
    pltpu.sync_copy(zeros_hbm, acc.at[pl.ds(s * INIT_PT, INIT_PT)])
    pltpu.sync_copy(dst_hbm.at[pl.ds(wid * ROWS_PT, ROWS_PT)], dst_v)
    pltpu.sync_copy(ones_hbm, ones_v)
    plsc.subcore_barrier()

    def body(j, carry):
        pltpu.sync_copy(ones_v, acc.at[dst_v.at[j]], add=True)
        return carry

    lax.fori_loop(0, ROWS_PT, body, 0)
    plsc.subcore_barrier()
    pltpu.sync_copy(
        acc.at[pl.ds(s * OUT_PT, OUT_PT)], out_hbm.at[c, pl.ds(s * OUT_PT, OUT_PT)]
    )


def _make_deg_kernel(interpret=False):
    return pl.kernel(
        _deg_body,
        out_type=jax.ShapeDtypeStruct((NC, N_ACC, D), jnp.float32),
        mesh=_MESH,
        scratch_types=[
            pltpu.VMEM((ROWS_PT, CHUNK), jnp.int32),
            pltpu.VMEM((CHUNK, D), jnp.float32),
            pltpu.VMEM_SHARED((N_ACC, D), jnp.float32),
        ],
        interpret=interpret,
    )


def _prop_body(g_hbm, src_hbm, dst_hbm, zeros_hbm, out_hbm, src_v, dst_v, rows_v, acc, sem):
    c = lax.axis_index("c")
    s = lax.axis_index("s")
    wid = c * NS + s
    pltpu.sync_copy(zeros_hbm, acc.at[pl.ds(s * INIT_PT, INIT_PT)])
    pltpu.sync_copy(src_hbm.at[pl.ds(wid * ROWS_PT, ROWS_PT)], src_v)
    pltpu.sync_copy(dst_hbm.at[pl.ds(wid * ROWS_PT, ROWS_PT)], dst_v)
    plsc.subcore_barrier()

    def body(j, carry):
        pltpu.async_copy(g_hbm.at[src_v.at[j]], rows_v, sem).wait()
        pltpu.sync_copy(rows_v, acc.at[dst_v.at[j]], add=True)
        return carry

    lax.fori_loop(0, ROWS_PT, body, 0)
    plsc.subcore_barrier()
    pltpu.sync_copy(
        acc.at[pl.ds(s * OUT_PT, OUT_PT)], out_hbm.at[c, pl.ds(s * OUT_PT, OUT_PT)]
    )


def _make_prop_kernel(interpret=False):
    return pl.kernel(
        _prop_body,
        out_type=jax.ShapeDtypeStruct((NC, N_ACC, D), jnp.float32),
        mesh=_MESH,
        scratch_types=[
            pltpu.VMEM((ROWS_PT, CHUNK), jnp.int32),
            pltpu.VMEM((ROWS_PT, CHUNK), jnp.int32),
            pltpu.VMEM((CHUNK, D), jnp.float32),
            pltpu.VMEM_SHARED((N_ACC, D), jnp.float32),
            pltpu.SemaphoreType.DMA,
        ],
        interpret=interpret,
    )


_deg_kernel = _make_deg_kernel()
_prop_kernel = _make_prop_kernel()


_R = 1000


def _k1_body(x_ref, degp_ref, w_ref, g_ref, dinv_ref):
    deg = degp_ref[0, :, :1] + degp_ref[1, :, :1] + 1.0
    dinv = lax.rsqrt(deg)
    h = jnp.dot(x_ref[...], w_ref[...], preferred_element_type=jnp.float32)
    g_ref[...] = h * dinv
    dinv_ref[...] = dinv


def _k1_call(x, degp, W1):
    grid = (N // _R,)
    return pl.pallas_call(
        _k1_body,
        grid=grid,
        in_specs=[
            pl.BlockSpec((_R, D), lambda i: (i, 0)),
            pl.BlockSpec((NC, _R, D), lambda i: (0, i, 0)),
            pl.BlockSpec((D, D), lambda i: (0, 0)),
        ],
        out_specs=[
            pl.BlockSpec((_R, D), lambda i: (i, 0)),
            pl.BlockSpec((_R, 1), lambda i: (i, 0)),
        ],
        out_shape=[
            jax.ShapeDtypeStruct((N, D), jnp.float32),
            jax.ShapeDtypeStruct((N, 1), jnp.float32),
        ],
    )(x, degp, W1)


def _k2_body(a_ref, g_ref, dinv_ref, b_ref, w_ref, out_ref):
    dinv = dinv_ref[...]
    z = (a_ref[0] + a_ref[1] + g_ref[...]) * dinv + b_ref[...]
    z = jnp.maximum(z, 0.0)
    out_ref[...] = jnp.dot(z, w_ref[...], preferred_element_type=jnp.float32) * dinv


def _k2_call(A, g1, dinv, b1, W2):
    grid = (N // _R,)
    return pl.pallas_call(
        _k2_body,
        grid=grid,
        in_specs=[
            pl.BlockSpec((NC, _R, D), lambda i: (0, i, 0)),
            pl.BlockSpec((_R, D), lambda i: (i, 0)),
            pl.BlockSpec((_R, 1), lambda i: (i, 0)),
            pl.BlockSpec((1, D), lambda i: (0, 0)),
            pl.BlockSpec((D, D), lambda i: (0, 0)),
        ],
        out_specs=pl.BlockSpec((_R, D), lambda i: (i, 0)),
        out_shape=jax.ShapeDtypeStruct((N, D), jnp.float32),
    )(A, g1, dinv, b1, W2)


def _k3_body(a_ref, g_ref, dinv_ref, b_ref, out_ref):
    out_ref[...] = (a_ref[0] + a_ref[1] + g_ref[...]) * dinv_ref[...] + b_ref[...]


def _k3_call(A, g2, dinv, b2):
    grid = (N // _R,)
    return pl.pallas_call(
        _k3_body,
        grid=grid,
        in_specs=[
            pl.BlockSpec((NC, _R, D), lambda i: (0, i, 0)),
            pl.BlockSpec((_R, D), lambda i: (i, 0)),
            pl.BlockSpec((_R, 1), lambda i: (i, 0)),
            pl.BlockSpec((1, D), lambda i: (0, 0)),
        ],
        out_specs=pl.BlockSpec((_R, D), lambda i: (i, 0)),
        out_shape=jax.ShapeDtypeStruct((N, D), jnp.float32),
    )(A, g2, dinv, b2)


def kernel(x, edge_index, W1, b1, W2, b2):
    src = edge_index[0].astype(jnp.int32)
    dst = edge_index[1].astype(jnp.int32)
    pad = E_PAD - E
    src_p = jnp.concatenate([src, jnp.zeros((pad,), jnp.int32)]).reshape(-1, CHUNK)
    dst_p = jnp.concatenate([dst, jnp.full((pad,), N, jnp.int32)]).reshape(-1, CHUNK)

    ones128 = jnp.ones((CHUNK, D), jnp.float32)
    zeros128 = jnp.zeros((INIT_PT, D), jnp.float32)

    degp = _deg_kernel(dst_p, ones128, zeros128)
    g1, dinv = _k1_call(x, degp, W1)
    A1 = _prop_kernel(g1, src_p, dst_p, zeros128)
    g2 = _k2_call(A1, g1, dinv, jnp.reshape(b1, (1, D)), W2)
    A2 = _prop_kernel(g2, src_p, dst_p, zeros128)
    out = _k3_call(A2, g2, dinv, jnp.reshape(b2, (1, D)))
    return out


# speedup vs baseline: 9.4512x; 1.0746x over previous
import jax
import jax.numpy as jnp
from jax import lax
from jax.experimental import pallas as pl
from jax.experimental.pallas import tpu as pltpu
from jax.experimental.pallas import tpu_sc as plsc

N = 10000
D = 128
E = 320000

NC = 2
NS = 16
NW = NC * NS

DCHUNK = 128
DROWS = 80
CHUNK = 128
PROWS = 80
E_PAD = NW * PROWS * CHUNK
N_ACC = 10240
INIT_PT = N_ACC // NS

_MESH = plsc.VectorSubcoreMesh(
    core_axis_name="c", subcore_axis_name="s", num_cores=NC, num_subcores=NS
)


def _deg_body(dst_hbm, ones_hbm, zeros_hbm, out_hbm, dst_v, ones_v, acc):
    c = lax.axis_index("c")
    s = lax.axis_index("s")
    wid = c * NS + s
    pltpu.sync_copy(zeros_hbm, acc.at[pl.ds(s * INIT_PT, INIT_PT)])
    pltpu.sync_copy(dst_hbm.at[pl.ds(wid * DROWS, DROWS)], dst_v)
    pltpu.sync_copy(ones_hbm, ones_v)
    plsc.subcore_barrier()

    def body(j, carry):
        pltpu.sync_copy(ones_v, acc.at[dst_v.at[j]], add=True)
        return carry

    lax.fori_loop(0, DROWS, body, 0)
    plsc.subcore_barrier()
    pltpu.sync_copy(
        acc.at[pl.ds(s * INIT_PT, INIT_PT)], out_hbm.at[c, pl.ds(s * INIT_PT, INIT_PT)]
    )


def _make_deg_kernel(interpret=False):
    return pl.kernel(
        _deg_body,
        out_type=jax.ShapeDtypeStruct((NC, N_ACC, D), jnp.float32),
        mesh=_MESH,
        scratch_types=[
            pltpu.VMEM((DROWS, DCHUNK), jnp.int32),
            pltpu.VMEM((DCHUNK, D), jnp.float32),
            pltpu.VMEM_SHARED((N_ACC, D), jnp.float32),
        ],
        interpret=interpret,
    )


def _fill(ref, nrows, value):
    vec = jnp.full((16,), value, jnp.float32)

    def row(r, carry):
        for k in range(D // 16):
            ref[r, pl.ds(k * 16, 16)] = vec
        return carry

    lax.fori_loop(0, nrows, row, 0)


_NBUF = 2
_NGRP = PROWS // _NBUF


def _prop_body(g_hbm, packed_hbm, out_hbm, packed_v, src_st, dst_st, *scratch):
    rows = scratch[:_NBUF]
    acc = scratch[_NBUF]
    zeros_hbm = None
    gsem = scratch[_NBUF + 1 : 2 * _NBUF + 1]
    ssem = scratch[2 * _NBUF + 1 :]
    c = lax.axis_index("c")
    s = lax.axis_index("s")
    wid = c * NS + s
    pltpu.sync_copy(packed_hbm.at[pl.ds(wid * PROWS, PROWS)], packed_v)
    _fill(rows[0], CHUNK, 0.0)
    for p in range(INIT_PT // CHUNK):
        pltpu.sync_copy(rows[0], acc.at[pl.ds(s * INIT_PT + p * CHUNK, CHUNK)])
    plsc.subcore_barrier()

    def unpack(jd, b):
        for k in range(D // 16):
            v = packed_v[jd, pl.ds(k * 16, 16)]
            src_st[b, pl.ds(k * 16, 16)] = jnp.bitwise_and(v, 0xFFFF)
            dst_st[b, pl.ds(k * 16, 16)] = jnp.right_shift(v, 16)

    def wait_scatter(b):
        pltpu.make_async_copy(rows[b], acc.at[dst_st.at[b]], ssem[b]).wait()

    def wait_gather(b):
        pltpu.make_async_copy(g_hbm.at[src_st.at[b]], rows[b], gsem[b]).wait()

    def issue_gather(b):
        pltpu.async_copy(g_hbm.at[src_st.at[b]], rows[b], gsem[b])

    def issue_scatter(b):
        pltpu.async_copy(rows[b], acc.at[dst_st.at[b]], ssem[b], add=True)

    unpack(0, 0)
    unpack(1, 1)
    issue_gather(0)
    issue_gather(1)
    wait_gather(0)
    issue_scatter(0)
    wait_scatter(0)
    unpack(2, 0)
    issue_gather(0)
    wait_gather(1)
    issue_scatter(1)

    def group(grp, carry):
        for b in range(_NBUF):
            j = grp * _NBUF + b
            bp = (b - 1) % _NBUF
            wait_scatter(bp)
            unpack(j + 1, bp)
            issue_gather(bp)
            wait_gather(b)
            issue_scatter(b)
        return carry

    lax.fori_loop(1, _NGRP - 1, group, 0)

    wait_scatter(1)
    unpack(PROWS - 1, 1)
    issue_gather(1)
    wait_gather(0)
    issue_scatter(0)
    wait_scatter(0)
    wait_gather(1)
    issue_scatter(1)
    wait_scatter(1)
    plsc.subcore_barrier()
    pltpu.sync_copy(
        acc.at[pl.ds(s * INIT_PT, INIT_PT)], out_hbm.at[c, pl.ds(s * INIT_PT, INIT_PT)]
    )


def _make_prop_kernel(interpret=False):
    return pl.kernel(
        _prop_body,
        out_type=jax.ShapeDtypeStruct((NC, N_ACC, D), jnp.float32),
        mesh=_MESH,
        scratch_types=[
            pltpu.VMEM((PROWS, CHUNK), jnp.int32),
            pltpu.VMEM((_NBUF, CHUNK), jnp.int32),
            pltpu.VMEM((_NBUF, CHUNK), jnp.int32),
        ]
        + [pltpu.VMEM((CHUNK, D), jnp.float32)] * _NBUF
        + [pltpu.VMEM_SHARED((N_ACC, D), jnp.float32)]
        + [pltpu.SemaphoreType.DMA] * (2 * _NBUF),
        interpret=interpret,
    )


_deg_kernel = _make_deg_kernel()
_prop_kernel = _make_prop_kernel()


_R = 1000


def _k1_body(x_ref, degp_ref, w_ref, g_ref, dinv_ref):
    deg = degp_ref[0, :, :1] + degp_ref[1, :, :1] + 1.0
    dinv = lax.rsqrt(deg)
    h = jnp.dot(x_ref[...], w_ref[...], preferred_element_type=jnp.float32)
    g_ref[...] = h * dinv
    dinv_ref[...] = dinv


def _k1_call(x, degp, W1):
    grid = (N // _R,)
    return pl.pallas_call(
        _k1_body,
        grid=grid,
        in_specs=[
            pl.BlockSpec((_R, D), lambda i: (i, 0)),
            pl.BlockSpec((NC, _R, D), lambda i: (0, i, 0)),
            pl.BlockSpec((D, D), lambda i: (0, 0)),
        ],
        out_specs=[
            pl.BlockSpec((_R, D), lambda i: (i, 0)),
            pl.BlockSpec((_R, 1), lambda i: (i, 0)),
        ],
        out_shape=[
            jax.ShapeDtypeStruct((N, D), jnp.float32),
            jax.ShapeDtypeStruct((N, 1), jnp.float32),
        ],
    )(x, degp, W1)


def _k2_body(a_ref, g_ref, dinv_ref, b_ref, w_ref, out_ref):
    dinv = dinv_ref[...]
    z = (a_ref[0] + a_ref[1] + g_ref[...]) * dinv + b_ref[...]
    z = jnp.maximum(z, 0.0)
    out_ref[...] = jnp.dot(z, w_ref[...], preferred_element_type=jnp.float32) * dinv


def _k2_call(A, g1, dinv, b1, W2):
    grid = (N // _R,)
    return pl.pallas_call(
        _k2_body,
        grid=grid,
        in_specs=[
            pl.BlockSpec((NC, _R, D), lambda i: (0, i, 0)),
            pl.BlockSpec((_R, D), lambda i: (i, 0)),
            pl.BlockSpec((_R, 1), lambda i: (i, 0)),
            pl.BlockSpec((1, D), lambda i: (0, 0)),
            pl.BlockSpec((D, D), lambda i: (0, 0)),
        ],
        out_specs=pl.BlockSpec((_R, D), lambda i: (i, 0)),
        out_shape=jax.ShapeDtypeStruct((N, D), jnp.float32),
    )(A, g1, dinv, b1, W2)


def _k3_body(a_ref, g_ref, dinv_ref, b_ref, out_ref):
    out_ref[...] = (a_ref[0] + a_ref[1] + g_ref[...]) * dinv_ref[...] + b_ref[...]


def _k3_call(A, g2, dinv, b2):
    grid = (N // _R,)
    return pl.pallas_call(
        _k3_body,
        grid=grid,
        in_specs=[
            pl.BlockSpec((NC, _R, D), lambda i: (0, i, 0)),
            pl.BlockSpec((_R, D), lambda i: (i, 0)),
            pl.BlockSpec((_R, 1), lambda i: (i, 0)),
            pl.BlockSpec((1, D), lambda i: (0, 0)),
        ],
        out_specs=pl.BlockSpec((_R, D), lambda i: (i, 0)),
        out_shape=jax.ShapeDtypeStruct((N, D), jnp.float32),
    )(A, g2, dinv, b2)


def kernel(x, edge_index, W1, b1, W2, b2):
    src = edge_index[0].astype(jnp.int32)
    dst = edge_index[1].astype(jnp.int32)
    pad = E_PAD - E
    src_f = jnp.concatenate([src, jnp.zeros((pad,), jnp.int32)])
    dst_f = jnp.concatenate([dst, jnp.full((pad,), N, jnp.int32)])
    dst_d = dst_f.reshape(-1, DCHUNK)
    packed = jnp.bitwise_or(src_f, jnp.left_shift(dst_f, 16)).reshape(-1, CHUNK)

    ones128 = jnp.ones((DCHUNK, D), jnp.float32)
    zeros128 = jnp.zeros((INIT_PT, D), jnp.float32)
    degp = _deg_kernel(dst_d, ones128, zeros128)
    g1, dinv = _k1_call(x, degp, W1)
    A1 = _prop_kernel(g1, packed)
    g2 = _k2_call(A1, g1, dinv, jnp.reshape(b1, (1, D)), W2)
    A2 = _prop_kernel(g2, packed)
    out = _k3_call(A2, g2, dinv, jnp.reshape(b2, (1, D)))
    return out
